# baseline (speedup 1.0000x reference)
import functools

import jax
import jax.numpy as jnp
from jax import lax
from jax.experimental import pallas as pl
from jax.experimental.pallas import tpu as pltpu
from jax.experimental.pallas import tpu_sc as plsc

D = 64
NW = 32
CHUNK = 128
SCALE = 1000.0


def _emb_body(n_chunks, lut_hbm, idx_hbm, out_hbm, idx_v, buf0, buf1,
              sem0, sem1):
    wid = lax.axis_index("s") * 2 + lax.axis_index("c")
    pltpu.sync_copy(idx_hbm.at[wid], idx_v)

    bufs = (buf0, buf1)
    sems = (sem0, sem1)

    def gather_start(j, b):
        pltpu.make_async_copy(lut_hbm.at[idx_v.at[j]], bufs[b], sems[b]).start()

    def finish_chunk(j, b):
        buf = bufs[b]
        pltpu.make_async_copy(lut_hbm.at[idx_v.at[j]], buf, sems[b]).wait()

        def scale_rows(i, carry):
            for r in range(4):
                for q in range(4):
                    sl = (i * 4 + r, pl.ds(q * 16, 16))
                    buf[sl] = buf[sl] * SCALE
            return carry

        lax.fori_loop(0, CHUNK // 4, scale_rows, 0)
        pltpu.sync_copy(buf, out_hbm.at[wid, j])

    gather_start(0, 0)

    def pair_body(j2, carry):
        j = j2 * 2
        gather_start(j + 1, 1)
        finish_chunk(j, 0)

        @pl.when(j + 2 < n_chunks)
        def _():
            gather_start(j + 2, 0)

        finish_chunk(j + 1, 1)
        return carry

    lax.fori_loop(0, n_chunks // 2, pair_body, 0)


@functools.partial(jax.jit, static_argnames=("n_chunks",))
def _emb_lookup(lut, idx, n_chunks):
    mesh = plsc.VectorSubcoreMesh(core_axis_name="c", subcore_axis_name="s")
    kern = pl.kernel(
        functools.partial(_emb_body, n_chunks),
        out_type=jax.ShapeDtypeStruct((NW, n_chunks, CHUNK, D), jnp.float32),
        mesh=mesh,
        scratch_types=[
            pltpu.VMEM((n_chunks, CHUNK), jnp.int32),
            pltpu.VMEM((CHUNK, D), jnp.float32),
            pltpu.VMEM((CHUNK, D), jnp.float32),
            pltpu.SemaphoreType.DMA,
            pltpu.SemaphoreType.DMA,
        ],
        compiler_params=pltpu.CompilerParams(use_tc_tiling_on_sc=False),
    )
    return kern(lut, idx)


def kernel(x, lut):
    b0, b1 = x.shape
    total = b0 * b1
    n_chunks = total // (NW * CHUNK)
    assert n_chunks * NW * CHUNK == total and n_chunks % 2 == 0
    idx = x.astype(jnp.int32).reshape(NW, n_chunks, CHUNK)
    out = _emb_lookup(lut, idx, n_chunks)
    return out.reshape(b0, b1, D)
